# baseline (speedup 1.0000x reference)
import functools

import jax
import jax.numpy as jnp
from jax import lax
from jax.experimental import pallas as pl
from jax.experimental.pallas import tpu as pltpu
from jax.experimental.pallas import tpu_sc as plsc


def _argmin_body(x_ref, w_ref, idx_ref):
    G = x_ref.shape[0]
    Wm = w_ref[...]
    E, D = Wm.shape
    w2 = jnp.sum(Wm * Wm, axis=1, keepdims=True)
    Wm2 = Wm + Wm
    eidx = lax.broadcasted_iota(jnp.int32, (E, 1), 0).astype(jnp.float32)
    for g in range(G):
        X = x_ref[g]
        a = jnp.sum(X * X, axis=0, keepdims=True)
        m2 = lax.dot_general(Wm2, X, (((1,), (0,)), ((), ())),
                             preferred_element_type=jnp.float32)
        d = (w2 + a) - m2
        dmin = jnp.min(d, axis=0, keepdims=True)
        cand = jnp.where(d == dmin, eidx, jnp.float32(jnp.inf))
        idx_ref[g, 0] = jnp.min(cand, axis=0).astype(jnp.int32)


def _argmin_indices(x, embeddings, group=4):
    B, C, N = x.shape
    E, D = embeddings.shape
    G = group
    return pl.pallas_call(
        _argmin_body,
        grid=(B // G,),
        in_specs=[
            pl.BlockSpec((G, C, N), lambda b: (b, 0, 0)),
            pl.BlockSpec((E, D), lambda b: (0, 0)),
        ],
        out_specs=pl.BlockSpec((G, 1, N), lambda b: (b, 0, 0)),
        out_shape=jax.ShapeDtypeStruct((B, 1, N), jnp.int32),
    )(x, embeddings)


def _sc_gather(table, idx_flat):
    E, D = table.shape
    (NB,) = idx_flat.shape
    info = plsc.get_sparse_core_info()
    NC, NS = info.num_cores, info.num_subcores
    NW = NC * NS
    b_per_w = NB // NW
    mesh = plsc.VectorSubcoreMesh(core_axis_name="c", subcore_axis_name="s")

    @functools.partial(
        pl.kernel,
        mesh=mesh,
        out_type=jax.ShapeDtypeStruct((NB, D), jnp.float32),
        scratch_types=[
            pltpu.VMEM((b_per_w,), jnp.int32),
            pltpu.VMEM((b_per_w, D), jnp.float32),
            pltpu.SemaphoreType.DMA,
        ],
        compiler_params=pltpu.CompilerParams(use_tc_tiling_on_sc=False),
    )
    def gather_k(table_hbm, idx_hbm, out_hbm, idx_v, rows_v, sem):
        wid = lax.axis_index("s") * NC + lax.axis_index("c")
        base = wid * b_per_w
        pltpu.sync_copy(idx_hbm.at[pl.ds(base, b_per_w)], idx_v)
        pltpu.async_copy(table_hbm.at[idx_v], rows_v, sem).wait()
        pltpu.sync_copy(rows_v, out_hbm.at[pl.ds(base, b_per_w)])

    return gather_k(table, idx_flat)


def kernel(input, embeddings):
    B, C, H, W = input.shape
    E, D = embeddings.shape
    N = H * W
    x = input.reshape(B, C, N)
    Bh = B // 2
    outs = []
    for h in range(2):
        xh = lax.slice_in_dim(x, h * Bh, (h + 1) * Bh, axis=0)
        idx = _argmin_indices(xh, embeddings)
        rows = _sc_gather(embeddings, idx.reshape(Bh * N))
        outs.append(rows.reshape(Bh, H, W, D).transpose(0, 3, 1, 2))
    return jnp.concatenate(outs, axis=0)


# speedup vs baseline: 1.0973x; 1.0973x over previous
import functools

import jax
import jax.numpy as jnp
from jax import lax
from jax.experimental import pallas as pl
from jax.experimental.pallas import tpu as pltpu
from jax.experimental.pallas import tpu_sc as plsc


def _argmin_body(x_ref, w_ref, idx_ref):
    G = x_ref.shape[0]
    Wm = w_ref[...]
    E, D = Wm.shape
    w2 = jnp.sum(Wm * Wm, axis=1, keepdims=True)
    Wm2 = Wm + Wm
    eidx = lax.broadcasted_iota(jnp.int32, (E, 1), 0).astype(jnp.float32)
    for g in range(G):
        X = x_ref[g]
        a = jnp.sum(X * X, axis=0, keepdims=True)
        m2 = lax.dot_general(Wm2, X, (((1,), (0,)), ((), ())),
                             preferred_element_type=jnp.float32)
        d = (w2 + a) - m2
        dmin = jnp.min(d, axis=0, keepdims=True)
        cand = jnp.where(d == dmin, eidx, jnp.float32(jnp.inf))
        idx_ref[g, 0] = jnp.min(cand, axis=0).astype(jnp.int32)


def _argmin_indices(x, embeddings, group=4):
    B, C, N = x.shape
    E, D = embeddings.shape
    G = group
    return pl.pallas_call(
        _argmin_body,
        grid=(B // G,),
        in_specs=[
            pl.BlockSpec((G, C, N), lambda b: (b, 0, 0)),
            pl.BlockSpec((E, D), lambda b: (0, 0)),
        ],
        out_specs=pl.BlockSpec((G, 1, N), lambda b: (b, 0, 0)),
        out_shape=jax.ShapeDtypeStruct((B, 1, N), jnp.int32),
        compiler_params=pltpu.CompilerParams(
            dimension_semantics=("parallel",)),
    )(x, embeddings)


def _sc_gather(table, idx_flat):
    E, D = table.shape
    (NB,) = idx_flat.shape
    info = plsc.get_sparse_core_info()
    NC, NS = info.num_cores, info.num_subcores
    NW = NC * NS
    b_per_w = NB // NW
    mesh = plsc.VectorSubcoreMesh(core_axis_name="c", subcore_axis_name="s")

    @functools.partial(
        pl.kernel,
        mesh=mesh,
        out_type=jax.ShapeDtypeStruct((NB, D), jnp.float32),
        scratch_types=[
            pltpu.VMEM((b_per_w,), jnp.int32),
            pltpu.VMEM((b_per_w, D), jnp.float32),
            pltpu.SemaphoreType.DMA,
        ],
        compiler_params=pltpu.CompilerParams(use_tc_tiling_on_sc=False),
    )
    def gather_k(table_hbm, idx_hbm, out_hbm, idx_v, rows_v, sem):
        wid = lax.axis_index("s") * NC + lax.axis_index("c")
        base = wid * b_per_w
        pltpu.sync_copy(idx_hbm.at[pl.ds(base, b_per_w)], idx_v)
        pltpu.async_copy(table_hbm.at[idx_v], rows_v, sem).wait()
        pltpu.sync_copy(rows_v, out_hbm.at[pl.ds(base, b_per_w)])

    return gather_k(table, idx_flat)


def kernel(input, embeddings):
    B, C, H, W = input.shape
    E, D = embeddings.shape
    N = H * W
    x = input.reshape(B, C, N)
    idx = _argmin_indices(x, embeddings)
    rows = _sc_gather(embeddings, idx.reshape(B * N))
    return rows.reshape(B, H, W, D).transpose(0, 3, 1, 2)


# speedup vs baseline: 1.1113x; 1.0128x over previous
import functools

import jax
import jax.numpy as jnp
from jax import lax
from jax.experimental import pallas as pl
from jax.experimental.pallas import tpu as pltpu
from jax.experimental.pallas import tpu_sc as plsc


def _argmin_body(x_ref, w_ref, idx_ref):
    G = x_ref.shape[0]
    Wm = w_ref[...]
    E, D = Wm.shape
    w2 = jnp.sum(Wm * Wm, axis=1, keepdims=True)
    Wm2 = Wm + Wm
    eidx = lax.broadcasted_iota(jnp.int32, (E, 1), 0).astype(jnp.float32)
    for g in range(G):
        X = x_ref[g]
        a = jnp.sum(X * X, axis=0, keepdims=True)
        m2 = lax.dot_general(Wm2, X, (((1,), (0,)), ((), ())),
                             preferred_element_type=jnp.float32)
        d = (w2 + a) - m2
        dmin = jnp.min(d, axis=0, keepdims=True)
        cand = jnp.where(d == dmin, eidx, jnp.float32(jnp.inf))
        N = X.shape[1]
        idx_ref[pl.ds(g * N, N)] = jnp.min(cand, axis=0).astype(jnp.int32)


def _argmin_indices(x, embeddings, group=4):
    B, C, N = x.shape
    E, D = embeddings.shape
    G = group
    return pl.pallas_call(
        _argmin_body,
        grid=(B // G,),
        in_specs=[
            pl.BlockSpec((G, C, N), lambda b: (b, 0, 0)),
            pl.BlockSpec((E, D), lambda b: (0, 0)),
        ],
        out_specs=pl.BlockSpec((G * N,), lambda b: (b,)),
        out_shape=jax.ShapeDtypeStruct((B * N,), jnp.int32),
    )(x, embeddings)


def _sc_gather(table, idx_flat):
    E, D = table.shape
    (NB,) = idx_flat.shape
    info = plsc.get_sparse_core_info()
    NC, NS = info.num_cores, info.num_subcores
    NW = NC * NS
    b_per_w = NB // NW
    mesh = plsc.VectorSubcoreMesh(core_axis_name="c", subcore_axis_name="s")

    @functools.partial(
        pl.kernel,
        mesh=mesh,
        out_type=jax.ShapeDtypeStruct((NB, D), jnp.float32),
        scratch_types=[
            pltpu.VMEM((b_per_w,), jnp.int32),
            pltpu.VMEM((b_per_w, D), jnp.float32),
            pltpu.SemaphoreType.DMA,
        ],
        compiler_params=pltpu.CompilerParams(use_tc_tiling_on_sc=False),
    )
    def gather_k(table_hbm, idx_hbm, out_hbm, idx_v, rows_v, sem):
        wid = lax.axis_index("s") * NC + lax.axis_index("c")
        base = wid * b_per_w
        pltpu.sync_copy(idx_hbm.at[pl.ds(base, b_per_w)], idx_v)
        pltpu.async_copy(table_hbm.at[idx_v], rows_v, sem).wait()
        pltpu.sync_copy(rows_v, out_hbm.at[pl.ds(base, b_per_w)])

    return gather_k(table, idx_flat)


def kernel(input, embeddings):
    B, C, H, W = input.shape
    E, D = embeddings.shape
    N = H * W
    x = input.reshape(B, C, N)
    idx = _argmin_indices(x, embeddings)
    rows = _sc_gather(embeddings, idx)
    return rows.reshape(B, H, W, D).transpose(0, 3, 1, 2)


# speedup vs baseline: 1.1165x; 1.0047x over previous
import functools

import jax
import jax.numpy as jnp
from jax import lax
from jax.experimental import pallas as pl
from jax.experimental.pallas import tpu as pltpu
from jax.experimental.pallas import tpu_sc as plsc


def _argmin_body(x_ref, w_ref, idx_ref):
    G = x_ref.shape[0]
    Wm = w_ref[...]
    E, D = Wm.shape
    w2 = jnp.sum(Wm * Wm, axis=1, keepdims=True)
    Wm2 = Wm + Wm
    eidx = lax.broadcasted_iota(jnp.int32, (E, 1), 0).astype(jnp.float32)
    for g in range(G):
        X = x_ref[g]
        a = jnp.sum(X * X, axis=0, keepdims=True)
        m2 = lax.dot_general(Wm2, X, (((1,), (0,)), ((), ())),
                             preferred_element_type=jnp.float32)
        d = (w2 + a) - m2
        dmin = jnp.min(d, axis=0, keepdims=True)
        cand = jnp.where(d == dmin, eidx, jnp.float32(jnp.inf))
        idx_ref[g, 0] = jnp.min(cand, axis=0).astype(jnp.int32)


def _argmin_indices(x, embeddings, group=4):
    B, C, N = x.shape
    E, D = embeddings.shape
    G = group
    return pl.pallas_call(
        _argmin_body,
        grid=(B // G,),
        in_specs=[
            pl.BlockSpec((G, C, N), lambda b: (b, 0, 0)),
            pl.BlockSpec((E, D), lambda b: (0, 0)),
        ],
        out_specs=pl.BlockSpec((G, 1, N), lambda b: (b, 0, 0)),
        out_shape=jax.ShapeDtypeStruct((B, 1, N), jnp.int32),
    )(x, embeddings)


def _sc_gather(table, idx_flat):
    E, D = table.shape
    (NB,) = idx_flat.shape
    info = plsc.get_sparse_core_info()
    NC, NS = info.num_cores, info.num_subcores
    NW = NC * NS
    b_per_w = NB // NW
    mesh = plsc.VectorSubcoreMesh(core_axis_name="c", subcore_axis_name="s")

    @functools.partial(
        pl.kernel,
        mesh=mesh,
        out_type=jax.ShapeDtypeStruct((NB, D), jnp.float32),
        scratch_types=[
            pltpu.VMEM((b_per_w,), jnp.int32),
            pltpu.VMEM((b_per_w, D), jnp.float32),
            pltpu.SemaphoreType.DMA,
        ],
        compiler_params=pltpu.CompilerParams(use_tc_tiling_on_sc=False),
    )
    def gather_k(table_hbm, idx_hbm, out_hbm, idx_v, rows_v, sem):
        wid = lax.axis_index("s") * NC + lax.axis_index("c")
        base = wid * b_per_w
        pltpu.sync_copy(idx_hbm.at[pl.ds(base, b_per_w)], idx_v)
        pltpu.async_copy(table_hbm.at[idx_v], rows_v, sem).wait()
        pltpu.sync_copy(rows_v, out_hbm.at[pl.ds(base, b_per_w)])

    return gather_k(table, idx_flat)


def kernel(input, embeddings):
    B, C, H, W = input.shape
    E, D = embeddings.shape
    N = H * W
    x = input.reshape(B, C, N)
    idx = _argmin_indices(x, embeddings)
    rows = _sc_gather(embeddings, idx.reshape(B * N))
    return rows.reshape(B, H, W, D).transpose(0, 3, 1, 2)
